# baseline (speedup 1.0000x reference)
import jax
import jax.numpy as jnp
from jax import lax
from jax.experimental import pallas as pl
from jax.experimental.pallas import tpu as pltpu
from jax.experimental.pallas import tpu_sc as plsc

N = 10000
E = 320000
F = 128
C = 64
D = 128
NPAD = 10016
DUMMY = N

NC = 2
NS = 16
NW = NC * NS
EPW = E // NW
BE = 80
STEPS = EPW // BE
NROWS = 10240
RPT = NROWS // NS
ZR = 32

BM1 = 2504
BM2 = 2000


def _mm_body(x_ref, w_ref, o_ref):
    xw = lax.dot_general(x_ref[...], w_ref[...],
                         (((1,), (1,)), ((), ())),
                         preferred_element_type=jnp.float32)
    i = pl.program_id(0)
    rows = i * BM1 + lax.broadcasted_iota(jnp.int32, (BM1, 1), 0)
    ones = (rows < N).astype(jnp.float32)
    o_ref[...] = jnp.concatenate(
        [xw, ones, jnp.zeros((BM1, D - C - 1), jnp.float32)], axis=1)


def _sc_body(xw_hbm, row_hbm, col_hbm, part_hbm,
             rfull, cfull, ab0, cb0, gb0, ab1, cb1, gb1, zbuf, acc,
             sg0, sg1, sz0, sz1):
    cid = lax.axis_index("c")
    tid = lax.axis_index("s")
    wid = cid * NS + tid
    r0 = tid * RPT
    ebase = wid * EPW

    pltpu.async_copy(row_hbm.at[pl.ds(ebase, EPW)], rfull, sg0)
    pltpu.async_copy(col_hbm.at[pl.ds(ebase, EPW)], cfull, sg1)

    def zfill(i, carry):
        for j in range(D // 16):
            zbuf[i, pl.ds(j * 16, 16)] = jnp.zeros((16,), jnp.float32)
        return carry
    lax.fori_loop(0, ZR, zfill, 0)
    zsem = (sz0, sz1)
    nz = RPT // ZR
    for k in range(nz):
        if k >= 2:
            pltpu.make_async_copy(
                zbuf, acc.at[pl.ds(r0 + (k - 2) * ZR, ZR)], zsem[k % 2]).wait()
        pltpu.async_copy(zbuf, acc.at[pl.ds(r0 + k * ZR, ZR)], zsem[k % 2])
    for k in (nz - 2, nz - 1):
        pltpu.make_async_copy(
            zbuf, acc.at[pl.ds(r0 + k * ZR, ZR)], zsem[k % 2]).wait()
    pltpu.make_async_copy(row_hbm.at[pl.ds(ebase, EPW)], rfull, sg0).wait()
    pltpu.make_async_copy(col_hbm.at[pl.ds(ebase, EPW)], cfull, sg1).wait()
    plsc.subcore_barrier()

    bufs = ((ab0, cb0, gb0, sg0), (ab1, cb1, gb1, sg1))

    def stage(s, p):
        ab, cb, gb, sg = bufs[p]
        off = s * BE
        for j in range(BE // 16):
            sl = pl.ds(off + j * 16, 16)
            dl = pl.ds(j * 16, 16)
            r = rfull[sl]
            c = cfull[sl]
            ab[dl] = jnp.where(r == c, jnp.full((16,), DUMMY, jnp.int32), r)
            cb[dl] = c
        pltpu.async_copy(xw_hbm.at[ab], gb, sg)

    def drain(p):
        ab, cb, gb, sg = bufs[p]
        pltpu.make_async_copy(xw_hbm.at[ab], gb, sg).wait()
        pltpu.sync_copy(gb, acc.at[cb], add=True)

    stage(0, 0)

    def body(i, carry):
        stage(2 * i + 1, 1)
        drain(0)
        stage(2 * i + 2, 0)
        drain(1)
        return carry
    lax.fori_loop(0, (STEPS - 1) // 2, body, 0)
    drain(0)

    plsc.subcore_barrier()
    pltpu.sync_copy(acc.at[pl.ds(r0, RPT)],
                    part_hbm.at[cid, pl.ds(r0, RPT)])


def _combine_body(p_ref, xw_ref, b_ref, o_ref):
    p = p_ref[0] + p_ref[1]
    num = p[:, :C] + xw_ref[:, :C]
    deg = p[:, C:C + 1] + 1.0
    o_ref[...] = num / deg + b_ref[...]


def kernel(x, adj, W, b):
    xp = jnp.pad(x, ((0, NPAD - N), (0, 0)))
    row = adj[0]
    col = adj[1]

    xw_aug = pl.pallas_call(
        _mm_body,
        grid=(NPAD // BM1,),
        in_specs=[
            pl.BlockSpec((BM1, F), lambda i: (i, 0)),
            pl.BlockSpec((C, F), lambda i: (0, 0)),
        ],
        out_specs=pl.BlockSpec((BM1, D), lambda i: (i, 0)),
        out_shape=jax.ShapeDtypeStruct((NPAD, D), jnp.float32),
    )(xp, W)

    mesh = plsc.VectorSubcoreMesh(core_axis_name="c", subcore_axis_name="s")
    part = pl.kernel(
        _sc_body,
        out_type=jax.ShapeDtypeStruct((NC, NROWS, D), jnp.float32),
        mesh=mesh,
        scratch_types=[
            pltpu.VMEM((EPW,), jnp.int32),
            pltpu.VMEM((EPW,), jnp.int32),
            pltpu.VMEM((BE,), jnp.int32),
            pltpu.VMEM((BE,), jnp.int32),
            pltpu.VMEM((BE, D), jnp.float32),
            pltpu.VMEM((BE,), jnp.int32),
            pltpu.VMEM((BE,), jnp.int32),
            pltpu.VMEM((BE, D), jnp.float32),
            pltpu.VMEM((ZR, D), jnp.float32),
            pltpu.VMEM_SHARED((NROWS, D), jnp.float32),
            pltpu.SemaphoreType.DMA,
            pltpu.SemaphoreType.DMA,
            pltpu.SemaphoreType.DMA,
            pltpu.SemaphoreType.DMA,
        ],
    )(xw_aug, row, col)

    out = pl.pallas_call(
        _combine_body,
        grid=(N // BM2,),
        in_specs=[
            pl.BlockSpec((NC, BM2, D), lambda i: (0, i, 0)),
            pl.BlockSpec((BM2, D), lambda i: (i, 0)),
            pl.BlockSpec((1, C), lambda i: (0, 0)),
        ],
        out_specs=pl.BlockSpec((BM2, C), lambda i: (i, 0)),
        out_shape=jax.ShapeDtypeStruct((N, C), jnp.float32),
    )(part, xw_aug, b.reshape(1, C))

    return out


# speedup vs baseline: 1.1380x; 1.1380x over previous
import jax
import jax.numpy as jnp
from jax import lax
from jax.experimental import pallas as pl
from jax.experimental.pallas import tpu as pltpu
from jax.experimental.pallas import tpu_sc as plsc

N = 10000
E = 320000
F = 128
C = 64
D = 128
NPAD = 10016
DUMMY = N

NC = 2
NS = 16
NW = NC * NS
EPW = E // NW
BE = 80
STEPS = EPW // BE
NROWS = 10240
RPT = NROWS // NS

BM1 = 2504
BM2 = 2000


def _mm_body(x_ref, w_ref, o_ref):
    xw = lax.dot_general(x_ref[...], w_ref[...],
                         (((1,), (1,)), ((), ())),
                         preferred_element_type=jnp.float32)
    i = pl.program_id(0)
    rows = i * BM1 + lax.broadcasted_iota(jnp.int32, (BM1, 1), 0)
    ones = (rows < N).astype(jnp.float32)
    o_ref[...] = jnp.concatenate(
        [xw, ones, jnp.zeros((BM1, D - C - 1), jnp.float32)], axis=1)


def _sc_body(xw_hbm, packed_hbm, part_hbm,
             pfull, ab0, cb0, gb0, ab1, cb1, gb1, ab2, cb2, gb2, acc,
             sg0, sg1, sg2, sz0, sz1):
    cid = lax.axis_index("c")
    tid = lax.axis_index("s")
    wid = cid * NS + tid
    r0 = tid * RPT
    ebase = wid * EPW

    pltpu.async_copy(packed_hbm.at[pl.ds(ebase, EPW)], pfull, sg0)

    def zfill(i, carry):
        for j in range(D // 16):
            gb0[i, pl.ds(j * 16, 16)] = jnp.zeros((16,), jnp.float32)
        return carry
    lax.fori_loop(0, BE, zfill, 0)
    zsem = (sz0, sz1)
    nz = RPT // BE
    for k in range(nz):
        if k >= 2:
            pltpu.make_async_copy(
                gb0, acc.at[pl.ds(r0 + (k - 2) * BE, BE)], zsem[k % 2]).wait()
        pltpu.async_copy(gb0, acc.at[pl.ds(r0 + k * BE, BE)], zsem[k % 2])
    for k in (nz - 2, nz - 1):
        pltpu.make_async_copy(
            gb0, acc.at[pl.ds(r0 + k * BE, BE)], zsem[k % 2]).wait()
    pltpu.make_async_copy(packed_hbm.at[pl.ds(ebase, EPW)], pfull, sg0).wait()
    plsc.subcore_barrier()

    bufs = ((ab0, cb0, gb0, sg0), (ab1, cb1, gb1, sg1), (ab2, cb2, gb2, sg2))

    def stage(s, p):
        ab, cb, gb, sg = bufs[p]
        off = s * BE
        for j in range(BE // 16):
            v = pfull[pl.ds(off + j * 16, 16)]
            dl = pl.ds(j * 16, 16)
            r = jnp.bitwise_and(v, jnp.full((16,), 0xFFFF, jnp.int32))
            c = lax.shift_right_logical(v, jnp.full((16,), 16, jnp.int32))
            ab[dl] = jnp.where(r == c, jnp.full((16,), DUMMY, jnp.int32), r)
            cb[dl] = c
        pltpu.async_copy(xw_hbm.at[ab], gb, sg)

    def drain(p):
        ab, cb, gb, sg = bufs[p]
        pltpu.make_async_copy(xw_hbm.at[ab], gb, sg).wait()
        pltpu.sync_copy(gb, acc.at[cb], add=True)

    stage(0, 0)
    stage(1, 1)

    def body(i, carry):
        stage(3 * i + 2, 2)
        drain(0)
        stage(3 * i + 3, 0)
        drain(1)
        stage(3 * i + 4, 1)
        drain(2)
        return carry
    lax.fori_loop(0, (STEPS - 2) // 3, body, 0)
    drain(0)
    drain(1)

    plsc.subcore_barrier()
    pltpu.sync_copy(acc.at[pl.ds(r0, RPT)],
                    part_hbm.at[cid, pl.ds(r0, RPT)])


def _combine_body(p_ref, xw_ref, b_ref, o_ref):
    p = p_ref[0] + p_ref[1]
    num = p[:, :C] + xw_ref[:, :C]
    deg = p[:, C:C + 1] + 1.0
    o_ref[...] = num / deg + b_ref[...]


def kernel(x, adj, W, b):
    xp = jnp.pad(x, ((0, NPAD - N), (0, 0)))
    packed = jnp.bitwise_or(adj[0], jnp.left_shift(adj[1], 16))

    xw_aug = pl.pallas_call(
        _mm_body,
        grid=(NPAD // BM1,),
        in_specs=[
            pl.BlockSpec((BM1, F), lambda i: (i, 0)),
            pl.BlockSpec((C, F), lambda i: (0, 0)),
        ],
        out_specs=pl.BlockSpec((BM1, D), lambda i: (i, 0)),
        out_shape=jax.ShapeDtypeStruct((NPAD, D), jnp.float32),
    )(xp, W)

    mesh = plsc.VectorSubcoreMesh(core_axis_name="c", subcore_axis_name="s")
    part = pl.kernel(
        _sc_body,
        out_type=jax.ShapeDtypeStruct((NC, NROWS, D), jnp.float32),
        mesh=mesh,
        scratch_types=[
            pltpu.VMEM((EPW,), jnp.int32),
            pltpu.VMEM((BE,), jnp.int32),
            pltpu.VMEM((BE,), jnp.int32),
            pltpu.VMEM((BE, D), jnp.float32),
            pltpu.VMEM((BE,), jnp.int32),
            pltpu.VMEM((BE,), jnp.int32),
            pltpu.VMEM((BE, D), jnp.float32),
            pltpu.VMEM((BE,), jnp.int32),
            pltpu.VMEM((BE,), jnp.int32),
            pltpu.VMEM((BE, D), jnp.float32),
            pltpu.VMEM_SHARED((NROWS, D), jnp.float32),
            pltpu.SemaphoreType.DMA,
            pltpu.SemaphoreType.DMA,
            pltpu.SemaphoreType.DMA,
            pltpu.SemaphoreType.DMA,
            pltpu.SemaphoreType.DMA,
        ],
    )(xw_aug, packed)

    out = pl.pallas_call(
        _combine_body,
        grid=(N // BM2,),
        in_specs=[
            pl.BlockSpec((NC, BM2, D), lambda i: (0, i, 0)),
            pl.BlockSpec((BM2, D), lambda i: (i, 0)),
            pl.BlockSpec((1, C), lambda i: (0, 0)),
        ],
        out_specs=pl.BlockSpec((BM2, C), lambda i: (i, 0)),
        out_shape=jax.ShapeDtypeStruct((N, C), jnp.float32),
    )(part, xw_aug, b.reshape(1, C))

    return out
